# baseline (speedup 1.0000x reference)
import functools

import jax
import jax.numpy as jnp
from jax import lax
from jax.experimental import pallas as pl
from jax.experimental.pallas import tpu as pltpu
from jax.experimental.pallas import tpu_sc as plsc

NC = 2
NS = 16
NW = NC * NS
CS = 2


@functools.partial(jax.jit, static_argnames=("V", "D", "S", "T"))
def _gather_rows(idx, table128, V, D, S, T):
    s_per_w = S // NW
    n_steps = s_per_w // CS
    mesh = plsc.VectorSubcoreMesh(core_axis_name="c", subcore_axis_name="s")

    @functools.partial(
        pl.kernel,
        out_type=jax.ShapeDtypeStruct((S, T, D), jnp.float32),
        mesh=mesh,
        scratch_types=[
            pltpu.VMEM((s_per_w, T), jnp.int32),
            pltpu.VMEM((2, CS * T, 2 * D), jnp.float32),
            pltpu.VMEM((2, CS, T, D), jnp.float32),
            pltpu.SemaphoreType.DMA,
            pltpu.SemaphoreType.DMA,
            pltpu.SemaphoreType.DMA,
            pltpu.SemaphoreType.DMA,
        ],
    )
    def k(idx_hbm, tab_hbm, out_hbm, idx_v, pair_v, half_v, g0, g1, w0, w1):
        wid = lax.axis_index("s") * NC + lax.axis_index("c")
        s_base = wid * s_per_w
        pltpu.sync_copy(idx_hbm.at[wid], idx_v)

        gsems = (g0, g1)
        wsems = (w0, w1)

        def gathers(i, b):
            for t in range(CS):
                pltpu.async_copy(
                    tab_hbm.at[idx_v.at[i * CS + t]],
                    pair_v.at[b].at[pl.ds(t * T, T)],
                    gsems[b],
                )

        def wait_gathers(b):
            for t in range(CS):
                pltpu.make_async_copy(
                    tab_hbm.at[idx_v.at[0]],
                    pair_v.at[b].at[pl.ds(t * T, T)],
                    gsems[b],
                ).wait()

        def compact(b):
            def row(g, _):
                t = g // T
                r = g % T
                for l in range(0, D, 16):
                    half_v.at[b].at[t][r, pl.ds(l, 16)] = (
                        pair_v.at[b][g, pl.ds(l, 16)]
                    )
                return 0

            lax.fori_loop(0, CS * T, row, 0)

        def write(i, b):
            pltpu.async_copy(
                half_v.at[b], out_hbm.at[pl.ds(s_base + i * CS, CS)], wsems[b]
            )

        def wait_write(b):
            pltpu.make_async_copy(
                half_v.at[b], out_hbm.at[pl.ds(s_base, CS)], wsems[b]
            ).wait()

        gathers(0, 0)
        gathers(1, 1)

        def body(i, _):
            for b in range(2):
                j = i * 2 + b
                wait_gathers(b)
                compact(b)
                write(j, b)
                wait_write(b)

                @pl.when(j + 2 < n_steps)
                def _():
                    gathers(j + 2, b)

            return 0

        lax.fori_loop(0, n_steps // 2, body, 0)

    return k(idx, table128)


def kernel(inputs, table):
    V, D = table.shape
    S, T = inputs.shape
    idx = inputs.reshape(NW, S // NW, T).astype(jnp.int32)
    table128 = jnp.pad(table, ((0, 0), (0, D)))
    return _gather_rows(idx, table128, V, D, S, T)


# speedup vs baseline: 1.1488x; 1.1488x over previous
import functools

import jax
import jax.numpy as jnp
from jax import lax
from jax.experimental import pallas as pl
from jax.experimental.pallas import tpu as pltpu
from jax.experimental.pallas import tpu_sc as plsc

NC = 2
NS = 16
NW = NC * NS
CS = 2


@functools.partial(jax.jit, static_argnames=("V", "D", "S", "T"))
def _gather_rows(idx, table128, V, D, S, T):
    s_per_w = S // NW
    n_steps = s_per_w // CS
    mesh = plsc.VectorSubcoreMesh(core_axis_name="c", subcore_axis_name="s")

    @functools.partial(
        pl.kernel,
        out_type=jax.ShapeDtypeStruct((S, T, D), jnp.float32),
        mesh=mesh,
        scratch_types=[
            pltpu.VMEM((s_per_w, T), jnp.int32),
            pltpu.VMEM((2, CS, T, 2 * D), jnp.float32),
            pltpu.VMEM((2, CS, T, D), jnp.float32),
            pltpu.SemaphoreType.DMA,
            pltpu.SemaphoreType.DMA,
            pltpu.SemaphoreType.DMA,
            pltpu.SemaphoreType.DMA,
        ],
    )
    def k(idx_hbm, tab_hbm, out_hbm, idx_v, pair_v, half_v, g0, g1, w0, w1):
        wid = lax.axis_index("s") * NC + lax.axis_index("c")
        s_base = wid * s_per_w
        pltpu.sync_copy(idx_hbm.at[wid], idx_v)

        gsems = (g0, g1)
        wsems = (w0, w1)

        def gathers(i, b):
            for t in range(CS):
                pltpu.async_copy(
                    tab_hbm.at[idx_v.at[i * CS + t]],
                    pair_v.at[b].at[t],
                    gsems[b],
                )

        def wait_gathers(b):
            for t in range(CS):
                pltpu.make_async_copy(
                    tab_hbm.at[idx_v.at[0]],
                    pair_v.at[b].at[t],
                    gsems[b],
                ).wait()

        def compact(b):
            for t in range(CS):
                for r in range(T):
                    for l in range(0, D, 16):
                        half_v.at[b].at[t][r, pl.ds(l, 16)] = (
                            pair_v.at[b].at[t][r, pl.ds(l, 16)]
                        )

        def write(i, b):
            pltpu.async_copy(
                half_v.at[b], out_hbm.at[pl.ds(s_base + i * CS, CS)], wsems[b]
            )

        def wait_write(b):
            pltpu.make_async_copy(
                half_v.at[b], out_hbm.at[pl.ds(s_base, CS)], wsems[b]
            ).wait()

        gathers(0, 0)
        gathers(1, 1)

        def body(i, _):
            for b in range(2):
                j = i * 2 + b
                wait_gathers(b)
                compact(b)
                write(j, b)
                wait_write(b)

                @pl.when(j + 2 < n_steps)
                def _():
                    gathers(j + 2, b)

            return 0

        lax.fori_loop(0, n_steps // 2, body, 0)

    return k(idx, table128)


def kernel(inputs, table):
    V, D = table.shape
    S, T = inputs.shape
    idx = inputs.reshape(NW, S // NW, T).astype(jnp.int32)
    table128 = jnp.pad(table, ((0, 0), (0, D)))
    return _gather_rows(idx, table128, V, D, S, T)
